# Initial kernel scaffold; baseline (speedup 1.0000x reference)
import jax
import jax.numpy as jnp
from jax.experimental import pallas as pl


def kernel(q, r, user_emb, item_emb, tW1, tb1, tW2, tb2, tW3, tb3, bW1, bb1, bW2, bb2, bW3, bb3, hW, hb):
    raise NotImplementedError("write your pallas kernel here")


# speedup vs baseline: 7.6958x; 7.6958x over previous
import functools

import jax
import jax.numpy as jnp
from jax import lax
from jax.experimental import pallas as pl
from jax.experimental.pallas import tpu as pltpu
from jax.experimental.pallas import tpu_sc as plsc

NUM_Q = 100000
EMB = 64
HID = 128
B = 4096
L = 50
TOKENS = B * L
ROWS128 = TOKENS // 128
BLK_ROWS = 1024
N_BLK = 98
PAD_ROWS = N_BLK * BLK_ROWS

NC = 2
NS = 16
NW = NC * NS
RPW = ROWS128 // NW


def _tables_body(u_ref, v_ref, tw1_ref, tb1_ref, w2t_ref,
                 bw1_ref, bb1_ref, bw2t_ref, sc_ref, tout_ref, bout_ref):
    u = u_ref[...]
    t1 = jnp.tanh(jnp.dot(u, tw1_ref[...],
                          preferred_element_type=jnp.float32) + tb1_ref[...])
    t2 = jnp.tanh(jnp.sum(t1 * w2t_ref[...], axis=1) + sc_ref[0])
    theta = t2 * sc_ref[1] + sc_ref[2]
    tout_ref[...] = theta.reshape(8, 128)

    v = v_ref[...]
    b1 = jnp.tanh(jnp.dot(v, bw1_ref[...],
                          preferred_element_type=jnp.float32) + bb1_ref[...])
    b2 = jnp.tanh(jnp.sum(b1 * bw2t_ref[...], axis=1) + sc_ref[3])
    beta = b2 * sc_ref[4] + sc_ref[5]
    bout_ref[...] = beta.reshape(8, 128)


def _compute_tables(user_emb, item_emb, tw1, tb1r, w2t, bw1, bb1r, bw2t, scal,
                    interpret=False):
    zero = lambda i: (0, 0)
    return pl.pallas_call(
        _tables_body,
        grid=(N_BLK,),
        in_specs=[
            pl.BlockSpec((BLK_ROWS, EMB), lambda i: (i, 0)),
            pl.BlockSpec((BLK_ROWS, EMB), lambda i: (i, 0)),
            pl.BlockSpec((EMB, HID), zero),
            pl.BlockSpec((1, HID), zero),
            pl.BlockSpec((1, HID), zero),
            pl.BlockSpec((EMB, HID), zero),
            pl.BlockSpec((1, HID), zero),
            pl.BlockSpec((1, HID), zero),
            pl.BlockSpec(memory_space=pltpu.SMEM),
        ],
        out_specs=[pl.BlockSpec((8, 128), lambda i: (i, 0)),
                   pl.BlockSpec((8, 128), lambda i: (i, 0))],
        out_shape=[jax.ShapeDtypeStruct((N_BLK * 8, 128), jnp.float32),
                   jax.ShapeDtypeStruct((N_BLK * 8, 128), jnp.float32)],
        interpret=interpret,
    )(user_emb, item_emb, tw1, tb1r, w2t, bw1, bb1r, bw2t, scal)


@functools.cache
def _gather_scalars_kernel():
    @functools.partial(
        pl.kernel,
        out_type=[jax.ShapeDtypeStruct((NW, RPW, 128), jnp.float32),
                  jax.ShapeDtypeStruct((NW, RPW, 128), jnp.float32)],
        mesh=plsc.VectorSubcoreMesh(core_axis_name="c", subcore_axis_name="s",
                                    num_cores=NC, num_subcores=NS),
        scratch_types=[
            pltpu.VMEM((RPW, 128), jnp.int32),
            pltpu.VMEM((RPW, 128), jnp.int32),
            pltpu.VMEM((RPW, 128), jnp.float32),
            pltpu.VMEM((RPW, 128), jnp.float32),
            pltpu.SemaphoreType.DMA,
            pltpu.SemaphoreType.DMA,
        ],
    )
    def _gather_scalars(ttab, btab, qi, bi, tg, bg,
                        qi_v, bi_v, rt_v, rb_v, s1, s2):
        wid = lax.axis_index("s") * NC + lax.axis_index("c")
        pltpu.sync_copy(qi.at[wid], qi_v)
        pltpu.sync_copy(bi.at[wid], bi_v)

        def body(j, carry):
            pltpu.async_copy(ttab.at[qi_v.at[j]], rt_v.at[j], s1).wait()
            pltpu.async_copy(btab.at[bi_v.at[j]], rb_v.at[j], s2).wait()
            return carry

        lax.fori_loop(0, RPW, body, 0)
        pltpu.sync_copy(rt_v, tg.at[wid])
        pltpu.sync_copy(rb_v, bg.at[wid])

    return _gather_scalars


def _softmax_body(hs_ref, tg_ref, bg_ref, o_ref):
    d = (tg_ref[...] - bg_ref[...]) * hs_ref[0] + hs_ref[1]
    m = jnp.max(d, axis=1, keepdims=True)
    e = jnp.exp(d - m)
    o_ref[...] = e / jnp.sum(e, axis=1, keepdims=True)


def _softmax(hs, tg, bg, interpret=False):
    return pl.pallas_call(
        _softmax_body,
        grid=(8,),
        in_specs=[
            pl.BlockSpec(memory_space=pltpu.SMEM),
            pl.BlockSpec((B // 8, L), lambda i: (i, 0)),
            pl.BlockSpec((B // 8, L), lambda i: (i, 0)),
        ],
        out_specs=pl.BlockSpec((B // 8, L), lambda i: (i, 0)),
        out_shape=jax.ShapeDtypeStruct((B, L), jnp.float32),
        interpret=interpret,
    )(hs, tg, bg)


def kernel(q, r, user_emb, item_emb, tW1, tb1, tW2, tb2, tW3, tb3,
           bW1, bb1, bW2, bb2, bW3, bb3, hW, hb):
    scal = jnp.concatenate([
        tb2.reshape(-1), tW3.reshape(-1), tb3.reshape(-1),
        bb2.reshape(-1), bW3.reshape(-1), bb3.reshape(-1),
    ]).astype(jnp.float32)
    ttab2d, btab2d = _compute_tables(
        user_emb, item_emb,
        tW1, tb1.reshape(1, HID), tW2.reshape(1, HID),
        bW1, bb1.reshape(1, HID), bW2.reshape(1, HID), scal)

    qi = q.astype(jnp.int32).reshape(NW, RPW, 128)
    bi = (q + r).astype(jnp.int32).reshape(NW, RPW, 128)
    tg, bg = _gather_scalars_kernel()(ttab2d.reshape(-1), btab2d.reshape(-1),
                                      qi, bi)

    hs = jnp.concatenate([hW.reshape(-1), hb.reshape(-1)]).astype(jnp.float32)
    return _softmax(hs, tg.reshape(B, L), bg.reshape(B, L))


# speedup vs baseline: 24.3197x; 3.1601x over previous
import functools

import jax
import jax.numpy as jnp
from jax import lax
from jax.experimental import pallas as pl
from jax.experimental.pallas import tpu as pltpu
from jax.experimental.pallas import tpu_sc as plsc

NUM_Q = 100000
EMB = 64
HID = 128
B = 4096
L = 50
TOKENS = B * L
ROWS128 = TOKENS // 128
BLK_ROWS = 1024
N_BLK = 98
PAD_ROWS = N_BLK * BLK_ROWS

NC = 2
NS = 16
NW = NC * NS
RPW = ROWS128 // NW


def _tables_body(ut_ref, vt_ref, tw1_ref, tb1_ref, w2c_ref,
                 bw1_ref, bb1_ref, bw2c_ref, sc_ref, tout_ref, bout_ref):
    dn = (((0,), (0,)), ((), ()))
    u = ut_ref[...]
    t1 = jnp.tanh(lax.dot_general(tw1_ref[...], u, dn,
                                  preferred_element_type=jnp.float32)
                  + tb1_ref[...])
    t2 = jnp.tanh(jnp.sum(t1 * w2c_ref[...], axis=0) + sc_ref[0])
    theta = t2 * sc_ref[1] + sc_ref[2]
    tout_ref[...] = theta.reshape(8, 128)

    v = vt_ref[...]
    b1 = jnp.tanh(lax.dot_general(bw1_ref[...], v, dn,
                                  preferred_element_type=jnp.float32)
                  + bb1_ref[...])
    b2 = jnp.tanh(jnp.sum(b1 * bw2c_ref[...], axis=0) + sc_ref[3])
    beta = b2 * sc_ref[4] + sc_ref[5]
    bout_ref[...] = beta.reshape(8, 128)


def _compute_tables(ut, vt, tw1, tb1c, w2c, bw1, bb1c, bw2c, scal,
                    interpret=False):
    zero = lambda i: (0, 0)
    return pl.pallas_call(
        _tables_body,
        grid=(N_BLK,),
        in_specs=[
            pl.BlockSpec((EMB, BLK_ROWS), lambda i: (0, i)),
            pl.BlockSpec((EMB, BLK_ROWS), lambda i: (0, i)),
            pl.BlockSpec((EMB, HID), zero),
            pl.BlockSpec((HID, 1), zero),
            pl.BlockSpec((HID, 1), zero),
            pl.BlockSpec((EMB, HID), zero),
            pl.BlockSpec((HID, 1), zero),
            pl.BlockSpec((HID, 1), zero),
            pl.BlockSpec(memory_space=pltpu.SMEM),
        ],
        out_specs=[pl.BlockSpec((8, 128), lambda i: (i, 0)),
                   pl.BlockSpec((8, 128), lambda i: (i, 0))],
        out_shape=[jax.ShapeDtypeStruct((N_BLK * 8, 128), jnp.float32),
                   jax.ShapeDtypeStruct((N_BLK * 8, 128), jnp.float32)],
        interpret=interpret,
    )(ut, vt, tw1, tb1c, w2c, bw1, bb1c, bw2c, scal)


@functools.cache
def _gather_scalars_kernel():
    @functools.partial(
        pl.kernel,
        out_type=[jax.ShapeDtypeStruct((NW, RPW, 128), jnp.float32),
                  jax.ShapeDtypeStruct((NW, RPW, 128), jnp.float32)],
        mesh=plsc.VectorSubcoreMesh(core_axis_name="c", subcore_axis_name="s",
                                    num_cores=NC, num_subcores=NS),
        scratch_types=[
            pltpu.VMEM((RPW, 128), jnp.int32),
            pltpu.VMEM((RPW, 128), jnp.int32),
            pltpu.VMEM((RPW, 128), jnp.float32),
            pltpu.VMEM((RPW, 128), jnp.float32),
            pltpu.SemaphoreType.DMA,
            pltpu.SemaphoreType.DMA,
        ],
    )
    def _gather_scalars(ttab, btab, qi, bi, tg, bg,
                        qi_v, bi_v, rt_v, rb_v, s1, s2):
        wid = lax.axis_index("s") * NC + lax.axis_index("c")
        pltpu.sync_copy(qi.at[wid], qi_v)
        pltpu.sync_copy(bi.at[wid], bi_v)

        def body(j, carry):
            pltpu.async_copy(ttab.at[qi_v.at[j]], rt_v.at[j], s1).wait()
            pltpu.async_copy(btab.at[bi_v.at[j]], rb_v.at[j], s2).wait()
            return carry

        lax.fori_loop(0, RPW, body, 0)
        pltpu.sync_copy(rt_v, tg.at[wid])
        pltpu.sync_copy(rb_v, bg.at[wid])

    return _gather_scalars


def _softmax_body(hs_ref, tg_ref, bg_ref, o_ref):
    d = (tg_ref[...] - bg_ref[...]) * hs_ref[0] + hs_ref[1]
    m = jnp.max(d, axis=0, keepdims=True)
    e = jnp.exp(d - m)
    o_ref[...] = e / jnp.sum(e, axis=0, keepdims=True)


def _softmax(hs, tg, bg, interpret=False):
    return pl.pallas_call(
        _softmax_body,
        grid=(8,),
        in_specs=[
            pl.BlockSpec(memory_space=pltpu.SMEM),
            pl.BlockSpec((L, B // 8), lambda i: (0, i)),
            pl.BlockSpec((L, B // 8), lambda i: (0, i)),
        ],
        out_specs=pl.BlockSpec((L, B // 8), lambda i: (0, i)),
        out_shape=jax.ShapeDtypeStruct((L, B), jnp.float32),
        interpret=interpret,
    )(hs, tg, bg)


def kernel(q, r, user_emb, item_emb, tW1, tb1, tW2, tb2, tW3, tb3,
           bW1, bb1, bW2, bb2, bW3, bb3, hW, hb):
    scal = jnp.concatenate([
        tb2.reshape(-1), tW3.reshape(-1), tb3.reshape(-1),
        bb2.reshape(-1), bW3.reshape(-1), bb3.reshape(-1),
    ]).astype(jnp.float32)
    ttab2d, btab2d = _compute_tables(
        user_emb.T, item_emb.T,
        tW1, tb1.reshape(HID, 1), tW2.reshape(HID, 1),
        bW1, bb1.reshape(HID, 1), bW2.reshape(HID, 1), scal)

    qt = q.astype(jnp.int32).T
    rt = r.astype(jnp.int32).T
    qi = qt.reshape(NW, RPW, 128)
    bi = (qt + rt).reshape(NW, RPW, 128)
    tg, bg = _gather_scalars_kernel()(ttab2d.reshape(-1), btab2d.reshape(-1),
                                      qi, bi)

    hs = jnp.concatenate([hW.reshape(-1), hb.reshape(-1)]).astype(jnp.float32)
    sm = _softmax(hs, tg.reshape(L, B), bg.reshape(L, B))
    return sm.T


# speedup vs baseline: 53.6061x; 2.2042x over previous
import functools

import jax
import jax.numpy as jnp
from jax import lax
from jax.experimental import pallas as pl
from jax.experimental.pallas import tpu as pltpu
from jax.experimental.pallas import tpu_sc as plsc

NUM_Q = 100000
EMB = 64
HID = 128
B = 4096
L = 50
TOKENS = B * L
ROWS128 = TOKENS // 128
BLK_ROWS = 7168
N_BLK = 14
PAD_ROWS = N_BLK * BLK_ROWS

NC = 2
NS = 16
NW = NC * NS
RPW = ROWS128 // NW


def _tables_body(ut_ref, vt_ref, tw1_ref, tb1_ref, w2c_ref,
                 bw1_ref, bb1_ref, bw2c_ref, sc_ref, tout_ref, bout_ref):
    dn = (((0,), (0,)), ((), ()))
    u = ut_ref[...]
    t1 = jnp.tanh(lax.dot_general(tw1_ref[...], u, dn,
                                  preferred_element_type=jnp.float32)
                  + tb1_ref[...])
    t2 = jnp.tanh(jnp.sum(t1 * w2c_ref[...], axis=0) + sc_ref[0])
    theta = t2 * sc_ref[1] + sc_ref[2]
    tout_ref[...] = theta.reshape(BLK_ROWS // 128, 128)

    v = vt_ref[...]
    b1 = jnp.tanh(lax.dot_general(bw1_ref[...], v, dn,
                                  preferred_element_type=jnp.float32)
                  + bb1_ref[...])
    b2 = jnp.tanh(jnp.sum(b1 * bw2c_ref[...], axis=0) + sc_ref[3])
    beta = b2 * sc_ref[4] + sc_ref[5]
    bout_ref[...] = beta.reshape(BLK_ROWS // 128, 128)


def _compute_tables(ut, vt, tw1, tb1c, w2c, bw1, bb1c, bw2c, scal,
                    interpret=False):
    zero = lambda i: (0, 0)
    return pl.pallas_call(
        _tables_body,
        grid=(N_BLK,),
        in_specs=[
            pl.BlockSpec((EMB, BLK_ROWS), lambda i: (0, i)),
            pl.BlockSpec((EMB, BLK_ROWS), lambda i: (0, i)),
            pl.BlockSpec((EMB, HID), zero),
            pl.BlockSpec((HID, 1), zero),
            pl.BlockSpec((HID, 1), zero),
            pl.BlockSpec((EMB, HID), zero),
            pl.BlockSpec((HID, 1), zero),
            pl.BlockSpec((HID, 1), zero),
            pl.BlockSpec(memory_space=pltpu.SMEM),
        ],
        out_specs=[pl.BlockSpec((BLK_ROWS // 128, 128), lambda i: (i, 0)),
                   pl.BlockSpec((BLK_ROWS // 128, 128), lambda i: (i, 0))],
        out_shape=[jax.ShapeDtypeStruct((PAD_ROWS // 128, 128), jnp.float32),
                   jax.ShapeDtypeStruct((PAD_ROWS // 128, 128), jnp.float32)],
        interpret=interpret,
    )(ut, vt, tw1, tb1c, w2c, bw1, bb1c, bw2c, scal)


@functools.cache
def _gather_scalars_kernel():
    @functools.partial(
        pl.kernel,
        out_type=[jax.ShapeDtypeStruct((NW, RPW, 128), jnp.float32),
                  jax.ShapeDtypeStruct((NW, RPW, 128), jnp.float32)],
        mesh=plsc.VectorSubcoreMesh(core_axis_name="c", subcore_axis_name="s",
                                    num_cores=NC, num_subcores=NS),
        scratch_types=[
            pltpu.VMEM((RPW, 128), jnp.int32),
            pltpu.VMEM((RPW, 128), jnp.int32),
            pltpu.VMEM((RPW, 128), jnp.float32),
            pltpu.VMEM((RPW, 128), jnp.float32),
            pltpu.SemaphoreType.DMA,
            pltpu.SemaphoreType.DMA,
        ],
    )
    def _gather_scalars(ttab, btab, qi, bi, tg, bg,
                        qi_v, bi_v, rt_v, rb_v, s1, s2):
        wid = lax.axis_index("s") * NC + lax.axis_index("c")
        pltpu.sync_copy(qi.at[wid], qi_v)
        pltpu.sync_copy(bi.at[wid], bi_v)

        def fire(j, carry):
            pltpu.async_copy(ttab.at[qi_v.at[j]], rt_v.at[j], s1)
            pltpu.async_copy(btab.at[bi_v.at[j]], rb_v.at[j], s2)
            return carry

        lax.fori_loop(0, RPW, fire, 0)

        def drain(j, carry):
            pltpu.make_async_copy(ttab.at[qi_v.at[j]], rt_v.at[j], s1).wait()
            pltpu.make_async_copy(btab.at[bi_v.at[j]], rb_v.at[j], s2).wait()
            return carry

        lax.fori_loop(0, RPW, drain, 0)
        pltpu.sync_copy(rt_v, tg.at[wid])
        pltpu.sync_copy(rb_v, bg.at[wid])

    return _gather_scalars


def _softmax_body(hs_ref, tg_ref, bg_ref, o_ref):
    d = (tg_ref[...] - bg_ref[...]) * hs_ref[0] + hs_ref[1]
    m = jnp.max(d, axis=0, keepdims=True)
    e = jnp.exp(d - m)
    o_ref[...] = e / jnp.sum(e, axis=0, keepdims=True)


def _softmax(hs, tg, bg, interpret=False):
    return pl.pallas_call(
        _softmax_body,
        grid=(8,),
        in_specs=[
            pl.BlockSpec(memory_space=pltpu.SMEM),
            pl.BlockSpec((L, B // 8), lambda i: (0, i)),
            pl.BlockSpec((L, B // 8), lambda i: (0, i)),
        ],
        out_specs=pl.BlockSpec((L, B // 8), lambda i: (0, i)),
        out_shape=jax.ShapeDtypeStruct((L, B), jnp.float32),
        interpret=interpret,
    )(hs, tg, bg)


def kernel(q, r, user_emb, item_emb, tW1, tb1, tW2, tb2, tW3, tb3,
           bW1, bb1, bW2, bb2, bW3, bb3, hW, hb):
    scal = jnp.concatenate([
        tb2.reshape(-1), tW3.reshape(-1), tb3.reshape(-1),
        bb2.reshape(-1), bW3.reshape(-1), bb3.reshape(-1),
    ]).astype(jnp.float32)
    ttab2d, btab2d = _compute_tables(
        user_emb.T, item_emb.T,
        tW1, tb1.reshape(HID, 1), tW2.reshape(HID, 1),
        bW1, bb1.reshape(HID, 1), bW2.reshape(HID, 1), scal)

    qt = q.astype(jnp.int32).T
    rt = r.astype(jnp.int32).T
    qi = qt.reshape(NW, RPW, 128)
    bi = (qt + rt).reshape(NW, RPW, 128)
    tg, bg = _gather_scalars_kernel()(ttab2d.reshape(-1), btab2d.reshape(-1),
                                      qi, bi)

    hs = jnp.concatenate([hW.reshape(-1), hb.reshape(-1)]).astype(jnp.float32)
    sm = _softmax(hs, tg.reshape(L, B), bg.reshape(L, B))
    return sm.T


# speedup vs baseline: 60.2468x; 1.1239x over previous
import functools

import jax
import jax.numpy as jnp
from jax import lax
from jax.experimental import pallas as pl
from jax.experimental.pallas import tpu as pltpu
from jax.experimental.pallas import tpu_sc as plsc

NUM_Q = 100000
EMB = 64
HID = 128
B = 4096
L = 50
TOKENS = B * L
BLK_ROWS = 7168
N_BLK = 14
PAD_ROWS = N_BLK * BLK_ROWS

NC = 2
NS = 16
NW = NC * NS


def _table_body(et_ref, w1_ref, aux_ref, b2_ref, w3_ref, b3_ref, out_ref):
    dn = (((0,), (0,)), ((), ()))
    b1c = aux_ref[:, 0:1]
    w2c = aux_ref[:, 1:2]
    e = et_ref[...]
    h1 = jnp.tanh(lax.dot_general(w1_ref[...], e, dn,
                                  preferred_element_type=jnp.float32)
                  + b1c)
    h2 = jnp.tanh(jnp.sum(h1 * w2c, axis=0) + b2_ref[0])
    out_ref[...] = (h2 * w3_ref[0] + b3_ref[0]).reshape(BLK_ROWS // 128, 128)


def _compute_table(et, w1, aux, b2, w3, b3, interpret=False):
    zero = lambda i: (0, 0)
    smem = pl.BlockSpec(memory_space=pltpu.SMEM)
    return pl.pallas_call(
        _table_body,
        grid=(N_BLK,),
        in_specs=[
            pl.BlockSpec((EMB, BLK_ROWS), lambda i: (0, i)),
            pl.BlockSpec((EMB, HID), zero),
            pl.BlockSpec((HID, 2), zero),
            smem, smem, smem,
        ],
        out_specs=pl.BlockSpec((BLK_ROWS // 128, 128), lambda i: (i, 0)),
        out_shape=jax.ShapeDtypeStruct((PAD_ROWS // 128, 128), jnp.float32),
        interpret=interpret,
    )(et, w1, aux, b2, w3, b3)


_SC_MESH = dict(core_axis_name="c", subcore_axis_name="s",
                num_cores=NC, num_subcores=NS)


@functools.cache
def _gather_theta_kernel():
    @functools.partial(
        pl.kernel,
        out_type=jax.ShapeDtypeStruct((L, B), jnp.float32),
        mesh=plsc.VectorSubcoreMesh(**_SC_MESH),
        scratch_types=[
            pltpu.VMEM((L, 128), jnp.int32),
            pltpu.VMEM((L, 128), jnp.float32),
            pltpu.SemaphoreType.DMA,
        ],
    )
    def _gather_theta(tab, qt, out, qi_v, rv_v, sem):
        wid = lax.axis_index("s") * NC + lax.axis_index("c")
        col = wid * 128
        pltpu.sync_copy(qt.at[:, pl.ds(col, 128)], qi_v)

        def fire(j, carry):
            pltpu.async_copy(tab.at[qi_v.at[j]], rv_v.at[j], sem)
            return carry

        lax.fori_loop(0, L, fire, 0)

        def drain(j, carry):
            pltpu.make_async_copy(tab.at[qi_v.at[j]], rv_v.at[j], sem).wait()
            return carry

        lax.fori_loop(0, L, drain, 0)
        pltpu.sync_copy(rv_v, out.at[:, pl.ds(col, 128)])

    return _gather_theta


@functools.cache
def _gather_beta_kernel():
    @functools.partial(
        pl.kernel,
        out_type=jax.ShapeDtypeStruct((L, B), jnp.float32),
        mesh=plsc.VectorSubcoreMesh(**_SC_MESH),
        scratch_types=[
            pltpu.VMEM((L, 128), jnp.int32),
            pltpu.VMEM((L, 128), jnp.int32),
            pltpu.VMEM((L, 128), jnp.float32),
            pltpu.SemaphoreType.DMA,
        ],
    )
    def _gather_beta(tab, qt, rt, out, qi_v, ri_v, rv_v, sem):
        wid = lax.axis_index("s") * NC + lax.axis_index("c")
        col = wid * 128
        pltpu.sync_copy(qt.at[:, pl.ds(col, 128)], qi_v)
        pltpu.sync_copy(rt.at[:, pl.ds(col, 128)], ri_v)

        def add_row(j, carry):
            def add_chunk(k, carry2):
                s = k * 16
                qi_v[j, pl.ds(s, 16)] = (qi_v[j, pl.ds(s, 16)]
                                         + ri_v[j, pl.ds(s, 16)])
                return carry2
            return lax.fori_loop(0, 8, add_chunk, carry)

        lax.fori_loop(0, L, add_row, 0)

        def fire(j, carry):
            pltpu.async_copy(tab.at[qi_v.at[j]], rv_v.at[j], sem)
            return carry

        lax.fori_loop(0, L, fire, 0)

        def drain(j, carry):
            pltpu.make_async_copy(tab.at[qi_v.at[j]], rv_v.at[j], sem).wait()
            return carry

        lax.fori_loop(0, L, drain, 0)
        pltpu.sync_copy(rv_v, out.at[:, pl.ds(col, 128)])

    return _gather_beta


def _softmax_body(hw_ref, hb_ref, tg_ref, bg_ref, o_ref):
    d = (tg_ref[...] - bg_ref[...]) * hw_ref[0] + hb_ref[0]
    m = jnp.max(d, axis=0, keepdims=True)
    e = jnp.exp(d - m)
    o_ref[...] = e / jnp.sum(e, axis=0, keepdims=True)


def _softmax(hw, hb, tg, bg, interpret=False):
    smem = pl.BlockSpec(memory_space=pltpu.SMEM)
    return pl.pallas_call(
        _softmax_body,
        grid=(8,),
        in_specs=[
            smem, smem,
            pl.BlockSpec((L, B // 8), lambda i: (0, i)),
            pl.BlockSpec((L, B // 8), lambda i: (0, i)),
        ],
        out_specs=pl.BlockSpec((L, B // 8), lambda i: (0, i)),
        out_shape=jax.ShapeDtypeStruct((L, B), jnp.float32),
        interpret=interpret,
    )(hw, hb, tg, bg)


def kernel(q, r, user_emb, item_emb, tW1, tb1, tW2, tb2, tW3, tb3,
           bW1, bb1, bW2, bb2, bW3, bb3, hW, hb):
    aux_t = jnp.stack([tb1, tW2[:, 0]], axis=1)
    aux_b = jnp.stack([bb1, bW2[:, 0]], axis=1)
    ttab2d = _compute_table(user_emb.T, tW1, aux_t, tb2.reshape(-1),
                            tW3.reshape(-1), tb3.reshape(-1))
    btab2d = _compute_table(item_emb.T, bW1, aux_b, bb2.reshape(-1),
                            bW3.reshape(-1), bb3.reshape(-1))

    qt = q.astype(jnp.int32).T
    rt = r.astype(jnp.int32).T
    tg = _gather_theta_kernel()(ttab2d.reshape(-1), qt)
    bg = _gather_beta_kernel()(btab2d.reshape(-1), qt, rt)

    sm = _softmax(hW.reshape(-1), hb.reshape(-1), tg, bg)
    return sm.T


# speedup vs baseline: 61.5496x; 1.0216x over previous
import functools

import jax
import jax.numpy as jnp
from jax import lax
from jax.experimental import pallas as pl
from jax.experimental.pallas import tpu as pltpu
from jax.experimental.pallas import tpu_sc as plsc

NUM_Q = 100000
EMB = 64
HID = 128
B = 4096
L = 50
TOKENS = B * L
BLK_ROWS = 7168
N_BLK = 14
PAD_ROWS = N_BLK * BLK_ROWS

NC = 2
NS = 16
NW = NC * NS
RPW = L


def _tables_body(ut_ref, vt_ref, tw1_ref, bw1_ref, aux_ref,
                 tb2_ref, tw3_ref, tb3_ref, bb2_ref, bw3_ref, bb3_ref,
                 tout_ref, bout_ref):
    dn = (((0,), (0,)), ((), ()))
    tb1c = aux_ref[:, 0:1]
    w2c = aux_ref[:, 1:2]
    bb1c = aux_ref[:, 2:3]
    bw2c = aux_ref[:, 3:4]

    u = ut_ref[...]
    t1 = jnp.tanh(lax.dot_general(tw1_ref[...], u, dn,
                                  preferred_element_type=jnp.float32)
                  + tb1c)
    t2 = jnp.tanh(jnp.sum(t1 * w2c, axis=0) + tb2_ref[0])
    theta = t2 * tw3_ref[0] + tb3_ref[0]
    tout_ref[...] = theta.reshape(BLK_ROWS // 128, 128)

    v = vt_ref[...]
    b1 = jnp.tanh(lax.dot_general(bw1_ref[...], v, dn,
                                  preferred_element_type=jnp.float32)
                  + bb1c)
    b2 = jnp.tanh(jnp.sum(b1 * bw2c, axis=0) + bb2_ref[0])
    beta = b2 * bw3_ref[0] + bb3_ref[0]
    bout_ref[...] = beta.reshape(BLK_ROWS // 128, 128)


def _compute_tables(ut, vt, tw1, bw1, aux,
                    tb2, tw3, tb3, bb2, bw3, bb3, interpret=False):
    zero = lambda i: (0, 0)
    smem = pl.BlockSpec(memory_space=pltpu.SMEM)
    return pl.pallas_call(
        _tables_body,
        grid=(N_BLK,),
        in_specs=[
            pl.BlockSpec((EMB, BLK_ROWS), lambda i: (0, i)),
            pl.BlockSpec((EMB, BLK_ROWS), lambda i: (0, i)),
            pl.BlockSpec((EMB, HID), zero),
            pl.BlockSpec((EMB, HID), zero),
            pl.BlockSpec((HID, 4), zero),
            smem, smem, smem, smem, smem, smem,
        ],
        out_specs=[pl.BlockSpec((BLK_ROWS // 128, 128), lambda i: (i, 0)),
                   pl.BlockSpec((BLK_ROWS // 128, 128), lambda i: (i, 0))],
        out_shape=[jax.ShapeDtypeStruct((PAD_ROWS // 128, 128), jnp.float32),
                   jax.ShapeDtypeStruct((PAD_ROWS // 128, 128), jnp.float32)],
        interpret=interpret,
    )(ut, vt, tw1, bw1, aux, tb2, tw3, tb3, bb2, bw3, bb3)


@functools.cache
def _gather_scalars_kernel():
    @functools.partial(
        pl.kernel,
        out_type=[jax.ShapeDtypeStruct((L, B), jnp.float32),
                  jax.ShapeDtypeStruct((L, B), jnp.float32)],
        mesh=plsc.VectorSubcoreMesh(core_axis_name="c", subcore_axis_name="s",
                                    num_cores=NC, num_subcores=NS),
        scratch_types=[
            pltpu.VMEM((RPW, 128), jnp.int32),
            pltpu.VMEM((RPW, 128), jnp.int32),
            pltpu.VMEM((RPW, 128), jnp.float32),
            pltpu.VMEM((RPW, 128), jnp.float32),
            pltpu.SemaphoreType.DMA,
            pltpu.SemaphoreType.DMA,
        ],
    )
    def _gather_scalars(ttab, btab, qt, bt, tg, bg,
                        qi_v, bi_v, rt_v, rb_v, s1, s2):
        wid = lax.axis_index("s") * NC + lax.axis_index("c")
        col = wid * 128
        pltpu.sync_copy(qt.at[:, pl.ds(col, 128)], qi_v)
        pltpu.sync_copy(bt.at[:, pl.ds(col, 128)], bi_v)

        def fire(j, carry):
            pltpu.async_copy(ttab.at[qi_v.at[j]], rt_v.at[j], s1)
            pltpu.async_copy(btab.at[bi_v.at[j]], rb_v.at[j], s2)
            return carry

        lax.fori_loop(0, RPW, fire, 0)

        def drain(j, carry):
            pltpu.make_async_copy(ttab.at[qi_v.at[j]], rt_v.at[j], s1).wait()
            pltpu.make_async_copy(btab.at[bi_v.at[j]], rb_v.at[j], s2).wait()
            return carry

        lax.fori_loop(0, RPW, drain, 0)
        pltpu.sync_copy(rt_v, tg.at[:, pl.ds(col, 128)])
        pltpu.sync_copy(rb_v, bg.at[:, pl.ds(col, 128)])

    return _gather_scalars


def _softmax_body(hw_ref, hb_ref, tg_ref, bg_ref, o_ref):
    d = (tg_ref[...] - bg_ref[...]) * hw_ref[0] + hb_ref[0]
    m = jnp.max(d, axis=0, keepdims=True)
    e = jnp.exp(d - m)
    o_ref[...] = e / jnp.sum(e, axis=0, keepdims=True)


def _softmax(hw, hb, tg, bg, interpret=False):
    smem = pl.BlockSpec(memory_space=pltpu.SMEM)
    return pl.pallas_call(
        _softmax_body,
        grid=(8,),
        in_specs=[
            smem, smem,
            pl.BlockSpec((L, B // 8), lambda i: (0, i)),
            pl.BlockSpec((L, B // 8), lambda i: (0, i)),
        ],
        out_specs=pl.BlockSpec((L, B // 8), lambda i: (0, i)),
        out_shape=jax.ShapeDtypeStruct((L, B), jnp.float32),
        interpret=interpret,
    )(hw, hb, tg, bg)


def kernel(q, r, user_emb, item_emb, tW1, tb1, tW2, tb2, tW3, tb3,
           bW1, bb1, bW2, bb2, bW3, bb3, hW, hb):
    aux = jnp.stack([tb1, tW2[:, 0], bb1, bW2[:, 0]], axis=1)
    ttab2d, btab2d = _compute_tables(
        user_emb.T, item_emb.T, tW1, bW1, aux,
        tb2.reshape(-1), tW3.reshape(-1), tb3.reshape(-1),
        bb2.reshape(-1), bW3.reshape(-1), bb3.reshape(-1))

    qt = q.astype(jnp.int32).T
    bt = qt + r.astype(jnp.int32).T
    tg, bg = _gather_scalars_kernel()(ttab2d.reshape(-1), btab2d.reshape(-1),
                                      qt, bt)

    sm = _softmax(hW.reshape(-1), hb.reshape(-1), tg, bg)
    return sm.T


# speedup vs baseline: 63.8266x; 1.0370x over previous
import functools

import jax
import jax.numpy as jnp
from jax import lax
from jax.experimental import pallas as pl
from jax.experimental.pallas import tpu as pltpu
from jax.experimental.pallas import tpu_sc as plsc

NUM_Q = 100000
EMB = 64
HID = 128
B = 4096
L = 50
TOKENS = B * L
BLK_ROWS = 14336
N_BLK = 7
PAD_ROWS = N_BLK * BLK_ROWS

NC = 2
NS = 16
NW = NC * NS
RPW = L


def _tables_body(ut_ref, vt_ref, tw1_ref, bw1_ref, aux_ref,
                 tb2_ref, tw3_ref, tb3_ref, bb2_ref, bw3_ref, bb3_ref,
                 tout_ref, bout_ref):
    dn = (((0,), (0,)), ((), ()))
    tb1c = aux_ref[:, 0:1]
    w2c = aux_ref[:, 1:2]
    bb1c = aux_ref[:, 2:3]
    bw2c = aux_ref[:, 3:4]

    u = ut_ref[...]
    t1 = jnp.tanh(lax.dot_general(tw1_ref[...], u, dn,
                                  preferred_element_type=jnp.float32)
                  + tb1c)
    t2 = jnp.tanh(jnp.sum(t1 * w2c, axis=0) + tb2_ref[0])
    theta = t2 * tw3_ref[0] + tb3_ref[0]
    tout_ref[...] = theta.reshape(BLK_ROWS // 128, 128)

    v = vt_ref[...]
    b1 = jnp.tanh(lax.dot_general(bw1_ref[...], v, dn,
                                  preferred_element_type=jnp.float32)
                  + bb1c)
    b2 = jnp.tanh(jnp.sum(b1 * bw2c, axis=0) + bb2_ref[0])
    beta = b2 * bw3_ref[0] + bb3_ref[0]
    bout_ref[...] = beta.reshape(BLK_ROWS // 128, 128)


def _compute_tables(ut, vt, tw1, bw1, aux,
                    tb2, tw3, tb3, bb2, bw3, bb3, interpret=False):
    zero = lambda i: (0, 0)
    smem = pl.BlockSpec(memory_space=pltpu.SMEM)
    return pl.pallas_call(
        _tables_body,
        grid=(N_BLK,),
        in_specs=[
            pl.BlockSpec((EMB, BLK_ROWS), lambda i: (0, i)),
            pl.BlockSpec((EMB, BLK_ROWS), lambda i: (0, i)),
            pl.BlockSpec((EMB, HID), zero),
            pl.BlockSpec((EMB, HID), zero),
            pl.BlockSpec((HID, 4), zero),
            smem, smem, smem, smem, smem, smem,
        ],
        out_specs=[pl.BlockSpec((BLK_ROWS // 128, 128), lambda i: (i, 0)),
                   pl.BlockSpec((BLK_ROWS // 128, 128), lambda i: (i, 0))],
        out_shape=[jax.ShapeDtypeStruct((PAD_ROWS // 128, 128), jnp.float32),
                   jax.ShapeDtypeStruct((PAD_ROWS // 128, 128), jnp.float32)],
        interpret=interpret,
    )(ut, vt, tw1, bw1, aux, tb2, tw3, tb3, bb2, bw3, bb3)


@functools.cache
def _gather_scalars_kernel():
    @functools.partial(
        pl.kernel,
        out_type=[jax.ShapeDtypeStruct((L, B), jnp.float32),
                  jax.ShapeDtypeStruct((L, B), jnp.float32)],
        mesh=plsc.VectorSubcoreMesh(core_axis_name="c", subcore_axis_name="s",
                                    num_cores=NC, num_subcores=NS),
        scratch_types=[
            pltpu.VMEM((RPW, 128), jnp.int32),
            pltpu.VMEM((RPW, 128), jnp.int32),
            pltpu.VMEM((RPW, 128), jnp.float32),
            pltpu.VMEM((RPW, 128), jnp.float32),
            pltpu.SemaphoreType.DMA,
            pltpu.SemaphoreType.DMA,
        ],
    )
    def _gather_scalars(ttab, btab, qt, bt, tg, bg,
                        qi_v, bi_v, rt_v, rb_v, s1, s2):
        wid = lax.axis_index("s") * NC + lax.axis_index("c")
        col = wid * 128
        pltpu.sync_copy(qt.at[:, pl.ds(col, 128)], qi_v)
        pltpu.sync_copy(bt.at[:, pl.ds(col, 128)], bi_v)

        def fire(j, carry):
            pltpu.async_copy(ttab.at[qi_v.at[j]], rt_v.at[j], s1)
            pltpu.async_copy(btab.at[bi_v.at[j]], rb_v.at[j], s2)
            return carry

        lax.fori_loop(0, RPW, fire, 0)

        def drain(j, carry):
            pltpu.make_async_copy(ttab.at[qi_v.at[j]], rt_v.at[j], s1).wait()
            pltpu.make_async_copy(btab.at[bi_v.at[j]], rb_v.at[j], s2).wait()
            return carry

        lax.fori_loop(0, RPW, drain, 0)
        pltpu.sync_copy(rt_v, tg.at[:, pl.ds(col, 128)])
        pltpu.sync_copy(rb_v, bg.at[:, pl.ds(col, 128)])

    return _gather_scalars


def _softmax_body(hw_ref, hb_ref, tg_ref, bg_ref, o_ref):
    d = (tg_ref[...] - bg_ref[...]) * hw_ref[0] + hb_ref[0]
    m = jnp.max(d, axis=0, keepdims=True)
    e = jnp.exp(d - m)
    o_ref[...] = e / jnp.sum(e, axis=0, keepdims=True)


def _softmax(hw, hb, tg, bg, interpret=False):
    smem = pl.BlockSpec(memory_space=pltpu.SMEM)
    return pl.pallas_call(
        _softmax_body,
        grid=(8,),
        in_specs=[
            smem, smem,
            pl.BlockSpec((L, B // 8), lambda i: (0, i)),
            pl.BlockSpec((L, B // 8), lambda i: (0, i)),
        ],
        out_specs=pl.BlockSpec((L, B // 8), lambda i: (0, i)),
        out_shape=jax.ShapeDtypeStruct((L, B), jnp.float32),
        interpret=interpret,
    )(hw, hb, tg, bg)


def kernel(q, r, user_emb, item_emb, tW1, tb1, tW2, tb2, tW3, tb3,
           bW1, bb1, bW2, bb2, bW3, bb3, hW, hb):
    aux = jnp.stack([tb1, tW2[:, 0], bb1, bW2[:, 0]], axis=1)
    ttab2d, btab2d = _compute_tables(
        user_emb.T, item_emb.T, tW1, bW1, aux,
        tb2.reshape(-1), tW3.reshape(-1), tb3.reshape(-1),
        bb2.reshape(-1), bW3.reshape(-1), bb3.reshape(-1))

    qt = q.astype(jnp.int32).T
    bt = qt + r.astype(jnp.int32).T
    tg, bg = _gather_scalars_kernel()(ttab2d.reshape(-1), btab2d.reshape(-1),
                                      qt, bt)

    sm = _softmax(hW.reshape(-1), hb.reshape(-1), tg, bg)
    return sm.T
